# Initial kernel scaffold; baseline (speedup 1.0000x reference)
import jax
import jax.numpy as jnp
from jax.experimental import pallas as pl


def kernel(x, edge_index, W1_l, b1_l, W1_r, W2_l, b2_l, W2_r):
    raise NotImplementedError("write your pallas kernel here")


# speedup vs baseline: 1.3570x; 1.3570x over previous
import functools

import jax
import jax.numpy as jnp
from jax import lax
from jax.experimental import pallas as pl
from jax.experimental.pallas import tpu as pltpu
from jax.experimental.pallas import tpu_sc as plsc

N = 10000
E = 320000
D = 128
NW = 32
R = 320
NPAD = NW * R
C = 3200
NCHUNK = E // C
GROUPS = C // 16
B = 128
SEL_PAD = C + 128


def _sc_segment_min(x, src, dst):
    mesh = plsc.VectorSubcoreMesh(core_axis_name="c", subcore_axis_name="s")

    @functools.partial(
        pl.kernel,
        mesh=mesh,
        out_type=jax.ShapeDtypeStruct((NPAD, D), jnp.float32),
        compiler_params=pltpu.CompilerParams(needs_layout_passes=False),
        scratch_types=[
            pltpu.VMEM((C,), jnp.int32),
            pltpu.VMEM((C,), jnp.int32),
            pltpu.VMEM((SEL_PAD,), jnp.int32),
            pltpu.VMEM((SEL_PAD,), jnp.int32),
            pltpu.VMEM((B, D), jnp.float32),
            pltpu.VMEM((R, D), jnp.float32),
            pltpu.SemaphoreType.DMA,
        ],
    )
    def k(x_hbm, src_hbm, dst_hbm, out_hbm, srcb, dstb, sel, dlc, msgs, agg, sem):
        w = lax.axis_index("s") * 2 + lax.axis_index("c")
        lo = w * R

        inf16 = jnp.full((16,), jnp.inf, jnp.float32)

        def init_row(r, carry):
            for cc in range(8):
                agg[r, pl.ds(cc * 16, 16)] = inf16
            return carry

        lax.fori_loop(0, R, init_row, 0)

        pad16 = jnp.full((16,), lo, jnp.int32)

        def chunk(t, carry):
            pltpu.sync_copy(src_hbm.at[pl.ds(t * C, C)], srcb)
            pltpu.sync_copy(dst_hbm.at[pl.ds(t * C, C)], dstb)

            def scan_group(g, cnt):
                d = dstb[pl.ds(g * 16, 16)]
                s = srcb[pl.ds(g * 16, 16)]
                dl = d - lo
                m = (dl >= 0) & (dl < R)
                mi = jnp.where(m, 1, 0)
                pos = cnt + plsc.cumsum(mi) - 1
                plsc.store_scatter(sel, [pos], s, mask=m)
                plsc.store_scatter(dlc, [pos], dl, mask=m)
                return cnt + jnp.sum(mi)

            cnt = lax.fori_loop(0, GROUPS, scan_group, 0)

            for kk in range(8):
                sel[pl.ds(cnt + kk * 16, 16)] = pad16

            nblk = (cnt + (B - 1)) // B

            lanes = lax.broadcasted_iota(jnp.int32, (16,), 0)

            def blk(b, carry2):
                pltpu.async_copy(x_hbm.at[sel.at[pl.ds(b * B, B)]], msgs, sem).wait()
                blen = jnp.minimum(cnt - b * B, B)

                def rmw(i, carry3):
                    gi = b * B + i
                    g0 = (gi >> 4) << 4
                    dl16 = dlc[pl.ds(g0, 16)]
                    drow = jnp.sum(jnp.where(lanes == (gi - g0), dl16, 0))
                    for cc in range(8):
                        slc = pl.ds(cc * 16, 16)
                        agg[drow, slc] = jnp.minimum(agg[drow, slc], msgs[i, slc])
                    return carry3

                lax.fori_loop(0, blen, rmw, 0)
                return carry2

            lax.fori_loop(0, nblk, blk, 0)
            return carry

        lax.fori_loop(0, NCHUNK, chunk, 0)
        pltpu.sync_copy(agg, out_hbm.at[pl.ds(lo, R)])

    return k(x, src, dst)


def _dense(agg, x, wl, b, wr, do_relu):
    BM = 1000

    def body(a_ref, x_ref, wl_ref, b_ref, wr_ref, o_ref):
        a = a_ref[...]
        a = jnp.where(jnp.isinf(a), 0.0, a)
        r = (
            jnp.dot(a, wl_ref[...], preferred_element_type=jnp.float32)
            + jnp.dot(x_ref[...], wr_ref[...], preferred_element_type=jnp.float32)
            + b_ref[...]
        )
        if do_relu:
            r = jnp.maximum(r, 0.0)
        o_ref[...] = r

    return pl.pallas_call(
        body,
        grid=(N // BM,),
        in_specs=[
            pl.BlockSpec((BM, D), lambda i: (i, 0)),
            pl.BlockSpec((BM, D), lambda i: (i, 0)),
            pl.BlockSpec((D, D), lambda i: (0, 0)),
            pl.BlockSpec((1, D), lambda i: (0, 0)),
            pl.BlockSpec((D, D), lambda i: (0, 0)),
        ],
        out_specs=pl.BlockSpec((BM, D), lambda i: (i, 0)),
        out_shape=jax.ShapeDtypeStruct((N, D), jnp.float32),
    )(agg, x, wl, b.reshape(1, D), wr)


def kernel(x, edge_index, W1_l, b1_l, W1_r, W2_l, b2_l, W2_r):
    src = edge_index[0]
    dst = edge_index[1]
    agg1 = _sc_segment_min(x, src, dst)[:N]
    h = _dense(agg1, x, W1_l, b1_l, W1_r, True)
    agg2 = _sc_segment_min(h, src, dst)[:N]
    out = _dense(agg2, h, W2_l, b2_l, W2_r, False)
    return out


# speedup vs baseline: 2.2381x; 1.6493x over previous
import functools

import jax
import jax.numpy as jnp
from jax import lax
from jax.experimental import pallas as pl
from jax.experimental.pallas import tpu as pltpu
from jax.experimental.pallas import tpu_sc as plsc

N = 10000
E = 320000
D = 128
NW = 32
R = 320
NPAD = NW * R
C = 3200
NCHUNK = E // C
GROUPS = C // 16
B = 128
SEL_PAD = C + 256
CAP = E + 2048

_SC_PARAMS = dict(
    compiler_params=pltpu.CompilerParams(needs_layout_passes=False),
)


def _al8(v):
    return pl.multiple_of(v, 8)


def _scan_edges(src, dst):
    mesh = plsc.VectorSubcoreMesh(core_axis_name="c", subcore_axis_name="s")

    @functools.partial(
        pl.kernel,
        mesh=mesh,
        out_type=(
            jax.ShapeDtypeStruct((NW * CAP,), jnp.int32),
            jax.ShapeDtypeStruct((NW * CAP,), jnp.int32),
            jax.ShapeDtypeStruct((NW * 16,), jnp.int32),
        ),
        scratch_types=[
            pltpu.VMEM((C,), jnp.int32),
            pltpu.VMEM((C,), jnp.int32),
            pltpu.VMEM((C,), jnp.int32),
            pltpu.VMEM((C,), jnp.int32),
            pltpu.VMEM((SEL_PAD,), jnp.int32),
            pltpu.VMEM((SEL_PAD,), jnp.int32),
            pltpu.VMEM((16,), jnp.int32),
            pltpu.SemaphoreType.DMA,
            pltpu.SemaphoreType.DMA,
        ],
        **_SC_PARAMS,
    )
    def k(src_hbm, dst_hbm, sel_hbm, dlc_hbm, cnt_hbm, srcb0, srcb1, dstb0,
          dstb1, sel, dlc, cbuf, sem_s, sem_d):
        w = lax.axis_index("s") * 2 + lax.axis_index("c")
        lo = w * R
        pad16 = jnp.full((16,), lo, jnp.int32)
        trash16 = jnp.full((16,), R, jnp.int32)

        def fire(t, sb, db):
            pltpu.async_copy(src_hbm.at[pl.ds(t * C, C)], sb, sem_s)
            pltpu.async_copy(dst_hbm.at[pl.ds(t * C, C)], db, sem_d)

        def drain(sb, db):
            pltpu.make_async_copy(src_hbm.at[pl.ds(0, C)], sb, sem_s).wait()
            pltpu.make_async_copy(dst_hbm.at[pl.ds(0, C)], db, sem_d).wait()

        fire(0, srcb0, dstb0)

        def chunk(t, off, sb, db, nsb, ndb):
            drain(sb, db)

            @pl.when(t + 1 < NCHUNK)
            def _():
                fire(t + 1, nsb, ndb)

            def scan_group(g, cv):
                d = db[pl.ds(g * 16, 16)]
                s = sb[pl.ds(g * 16, 16)]
                dl = d - lo
                m = (dl >= 0) & (dl < R)
                mi = jnp.where(m, 1, 0)
                pos = cv + plsc.cumsum(mi) - 1
                plsc.store_scatter(sel, [pos], s, mask=m)
                plsc.store_scatter(dlc, [pos], dl, mask=m)
                return cv + plsc.all_reduce_population_count(m)

            cv = lax.fori_loop(0, GROUPS, scan_group, jnp.zeros((16,), jnp.int32))
            cnt = jnp.sum(cv) >> 4
            for kk in range(8):
                sel[pl.ds(cnt + kk * 16, 16)] = pad16
                dlc[pl.ds(cnt + kk * 16, 16)] = trash16
            cnt16 = ((cnt + 15) >> 4) << 4
            nwr = (cnt16 + 127) >> 7

            def wr(kb, carry):
                pltpu.sync_copy(sel.at[pl.ds(kb * 128, 128)],
                                sel_hbm.at[pl.ds(_al8(w * CAP + off + kb * 128), 128)])
                pltpu.sync_copy(dlc.at[pl.ds(kb * 128, 128)],
                                dlc_hbm.at[pl.ds(_al8(w * CAP + off + kb * 128), 128)])
                return carry

            lax.fori_loop(0, nwr, wr, 0)
            return off + cnt16

        def pair(p, off):
            off = chunk(2 * p, off, srcb0, dstb0, srcb1, dstb1)
            off = chunk(2 * p + 1, off, srcb1, dstb1, srcb0, dstb0)
            return off

        off = lax.fori_loop(0, NCHUNK // 2, pair, 0)

        for kk in range(8):
            sel[pl.ds(kk * 16, 16)] = pad16
            dlc[pl.ds(kk * 16, 16)] = trash16
        pltpu.sync_copy(sel.at[pl.ds(0, 128)], sel_hbm.at[pl.ds(_al8(w * CAP + off), 128)])
        pltpu.sync_copy(dlc.at[pl.ds(0, 128)], dlc_hbm.at[pl.ds(_al8(w * CAP + off), 128)])
        cbuf[pl.ds(0, 16)] = jnp.full((16,), 1, jnp.int32) * off
        pltpu.sync_copy(cbuf, cnt_hbm.at[pl.ds(_al8(w * 16), 16)])

    return k(src, dst)


def _sc_agg(x, sel_all, dlc_all, counts):
    mesh = plsc.VectorSubcoreMesh(core_axis_name="c", subcore_axis_name="s")

    @functools.partial(
        pl.kernel,
        mesh=mesh,
        out_type=jax.ShapeDtypeStruct((NPAD, D), jnp.float32),
        scratch_types=[
            pltpu.VMEM((B,), jnp.int32),
            pltpu.VMEM((B,), jnp.int32),
            pltpu.VMEM((B,), jnp.int32),
            pltpu.VMEM((B,), jnp.int32),
            pltpu.VMEM((B, D), jnp.float32),
            pltpu.VMEM((B, D), jnp.float32),
            pltpu.VMEM((R + 1, D), jnp.float32),
            pltpu.VMEM((16,), jnp.int32),
            pltpu.SemaphoreType.DMA,
            pltpu.SemaphoreType.DMA,
        ],
        **_SC_PARAMS,
    )
    def k(x_hbm, sel_hbm, dlc_hbm, cnt_hbm, out_hbm, selv0, selv1, dlcv0,
          dlcv1, msgs0, msgs1, agg, cbuf, sem_i, sem_g):
        w = lax.axis_index("s") * 2 + lax.axis_index("c")
        lo = w * R
        lanes = lax.broadcasted_iota(jnp.int32, (16,), 0)
        inf16 = jnp.full((16,), jnp.inf, jnp.float32)

        def init_row(r, carry):
            for cc in range(8):
                agg[r, pl.ds(cc * 16, 16)] = inf16
            return carry

        lax.fori_loop(0, R + 1, init_row, 0)

        pltpu.sync_copy(cnt_hbm.at[pl.ds(_al8(w * 16), 16)], cbuf)
        tot = jnp.sum(cbuf[pl.ds(0, 16)]) >> 4
        nblk = jnp.maximum((tot + 127) >> 7, 1)

        def fire_idx(b, sv, dv):
            pltpu.async_copy(sel_hbm.at[pl.ds(_al8(w * CAP + b * B), B)], sv, sem_i)
            pltpu.async_copy(dlc_hbm.at[pl.ds(_al8(w * CAP + b * B), B)], dv, sem_i)

        def wait_idx(sv, dv):
            pltpu.make_async_copy(sel_hbm.at[pl.ds(0, B)], sv, sem_i).wait()
            pltpu.make_async_copy(dlc_hbm.at[pl.ds(0, B)], dv, sem_i).wait()

        def fire_gather(sv, mb):
            pltpu.async_copy(x_hbm.at[sv], mb, sem_g)

        def wait_gather(sv, mb):
            pltpu.make_async_copy(x_hbm.at[sv], mb, sem_g).wait()

        pltpu.sync_copy(sel_hbm.at[pl.ds(_al8(w * CAP), B)], selv0)
        pltpu.sync_copy(dlc_hbm.at[pl.ds(_al8(w * CAP), B)], dlcv0)
        fire_gather(selv0, msgs0)

        @pl.when(1 < nblk)
        def _():
            fire_idx(1, selv1, dlcv1)

        def half(b, sv, dv, mb, nsv, ndv, nmb):
            wait_gather(sv, mb)

            @pl.when(b + 1 < nblk)
            def _():
                wait_idx(nsv, ndv)
                fire_gather(nsv, nmb)

            db = dv

            def rmw(i, carry):
                g0 = (i >> 4) << 4
                dl16 = db[pl.ds(g0, 16)]
                drow = jnp.sum(jnp.where(lanes == i - g0, dl16, 0))
                for cc in range(8):
                    slc = pl.ds(cc * 16, 16)
                    agg[drow, slc] = jnp.minimum(agg[drow, slc], mb[i, slc])
                return carry

            lax.fori_loop(0, B, rmw, 0)

            @pl.when(b + 2 < nblk)
            def _():
                fire_idx(b + 2, sv, dv)

        def pairb(p, carry):
            b0 = 2 * p

            @pl.when(b0 < nblk)
            def _():
                half(b0, selv0, dlcv0, msgs0, selv1, dlcv1, msgs1)

            @pl.when(b0 + 1 < nblk)
            def _():
                half(b0 + 1, selv1, dlcv1, msgs1, selv0, dlcv0, msgs0)

            return carry

        lax.fori_loop(0, (nblk + 1) >> 1, pairb, 0)
        pltpu.sync_copy(agg.at[pl.ds(0, R)], out_hbm.at[pl.ds(lo, R)])

    return k(x, sel_all, dlc_all, counts)


def _dense(agg, x, wl, b, wr, do_relu):
    BM = 1000

    def body(a_ref, x_ref, wl_ref, b_ref, wr_ref, o_ref):
        a = a_ref[...]
        a = jnp.where(jnp.isinf(a), 0.0, a)
        r = (
            jnp.dot(a, wl_ref[...], preferred_element_type=jnp.float32)
            + jnp.dot(x_ref[...], wr_ref[...], preferred_element_type=jnp.float32)
            + b_ref[...]
        )
        if do_relu:
            r = jnp.maximum(r, 0.0)
        o_ref[...] = r

    return pl.pallas_call(
        body,
        grid=(N // BM,),
        in_specs=[
            pl.BlockSpec((BM, D), lambda i: (i, 0)),
            pl.BlockSpec((BM, D), lambda i: (i, 0)),
            pl.BlockSpec((D, D), lambda i: (0, 0)),
            pl.BlockSpec((1, D), lambda i: (0, 0)),
            pl.BlockSpec((D, D), lambda i: (0, 0)),
        ],
        out_specs=pl.BlockSpec((BM, D), lambda i: (i, 0)),
        out_shape=jax.ShapeDtypeStruct((N, D), jnp.float32),
    )(agg, x, wl, b.reshape(1, D), wr)


def kernel(x, edge_index, W1_l, b1_l, W1_r, W2_l, b2_l, W2_r):
    src = edge_index[0]
    dst = edge_index[1]
    sel_all, dlc_all, counts = _scan_edges(src, dst)
    agg1 = _sc_agg(x, sel_all, dlc_all, counts)[:N]
    h = _dense(agg1, x, W1_l, b1_l, W1_r, True)
    agg2 = _sc_agg(h, sel_all, dlc_all, counts)[:N]
    out = _dense(agg2, h, W2_l, b2_l, W2_r, False)
    return out


# speedup vs baseline: 2.9842x; 1.3334x over previous
import functools

import jax
import jax.numpy as jnp
from jax import lax
from jax.experimental import pallas as pl
from jax.experimental.pallas import tpu as pltpu
from jax.experimental.pallas import tpu_sc as plsc

N = 10000
E = 320000
D = 128
NW = 32
R = 320
NPAD = NW * R
C = 3200
NCHUNK = E // C
GROUPS = C // 16
B = 128
SEL_PAD = C + 256
CAP = E + 2048

_SC_PARAMS = dict(
    compiler_params=pltpu.CompilerParams(needs_layout_passes=False),
)


def _al8(v):
    return pl.multiple_of(v, 8)


def _scan_edges(src, dst):
    mesh = plsc.VectorSubcoreMesh(core_axis_name="c", subcore_axis_name="s")

    @functools.partial(
        pl.kernel,
        mesh=mesh,
        out_type=(
            jax.ShapeDtypeStruct((NW * CAP,), jnp.int32),
            jax.ShapeDtypeStruct((NW * CAP,), jnp.int32),
            jax.ShapeDtypeStruct((NW * 16,), jnp.int32),
        ),
        scratch_types=[
            pltpu.VMEM((C,), jnp.int32),
            pltpu.VMEM((C,), jnp.int32),
            pltpu.VMEM((C,), jnp.int32),
            pltpu.VMEM((C,), jnp.int32),
            pltpu.VMEM((SEL_PAD,), jnp.int32),
            pltpu.VMEM((SEL_PAD,), jnp.int32),
            pltpu.VMEM((16,), jnp.int32),
            pltpu.SemaphoreType.DMA,
            pltpu.SemaphoreType.DMA,
        ],
        **_SC_PARAMS,
    )
    def k(src_hbm, dst_hbm, sel_hbm, dlc_hbm, cnt_hbm, srcb0, srcb1, dstb0,
          dstb1, sel, dlc, cbuf, sem_s, sem_d):
        w = lax.axis_index("s") * 2 + lax.axis_index("c")
        lo = w * R
        pad16 = jnp.full((16,), lo, jnp.int32)
        trash16 = jnp.full((16,), R, jnp.int32)

        def fire(t, sb, db):
            pltpu.async_copy(src_hbm.at[pl.ds(t * C, C)], sb, sem_s)
            pltpu.async_copy(dst_hbm.at[pl.ds(t * C, C)], db, sem_d)

        def drain(sb, db):
            pltpu.make_async_copy(src_hbm.at[pl.ds(0, C)], sb, sem_s).wait()
            pltpu.make_async_copy(dst_hbm.at[pl.ds(0, C)], db, sem_d).wait()

        fire(0, srcb0, dstb0)

        def chunk(t, off, sb, db, nsb, ndb):
            drain(sb, db)

            @pl.when(t + 1 < NCHUNK)
            def _():
                fire(t + 1, nsb, ndb)

            def scan_group(g, cv):
                d = db[pl.ds(g * 16, 16)]
                s = sb[pl.ds(g * 16, 16)]
                dl = d - lo
                m = (dl >= 0) & (dl < R)
                mi = jnp.where(m, 1, 0)
                pos = cv + plsc.cumsum(mi) - 1
                plsc.store_scatter(sel, [pos], s, mask=m)
                plsc.store_scatter(dlc, [pos], dl, mask=m)
                return cv + plsc.all_reduce_population_count(m)

            cv = lax.fori_loop(0, GROUPS, scan_group, jnp.zeros((16,), jnp.int32))
            cnt = jnp.sum(cv) >> 4
            for kk in range(8):
                sel[pl.ds(cnt + kk * 16, 16)] = pad16
                dlc[pl.ds(cnt + kk * 16, 16)] = trash16
            cnt16 = ((cnt + 15) >> 4) << 4
            nwr = (cnt16 + 127) >> 7

            def wr(kb, carry):
                pltpu.sync_copy(sel.at[pl.ds(kb * 128, 128)],
                                sel_hbm.at[pl.ds(_al8(w * CAP + off + kb * 128), 128)])
                pltpu.sync_copy(dlc.at[pl.ds(kb * 128, 128)],
                                dlc_hbm.at[pl.ds(_al8(w * CAP + off + kb * 128), 128)])
                return carry

            lax.fori_loop(0, nwr, wr, 0)
            return off + cnt16

        def pair(p, off):
            off = chunk(2 * p, off, srcb0, dstb0, srcb1, dstb1)
            off = chunk(2 * p + 1, off, srcb1, dstb1, srcb0, dstb0)
            return off

        off = lax.fori_loop(0, NCHUNK // 2, pair, 0)

        for kk in range(8):
            sel[pl.ds(kk * 16, 16)] = pad16
            dlc[pl.ds(kk * 16, 16)] = trash16
        pltpu.sync_copy(sel.at[pl.ds(0, 128)], sel_hbm.at[pl.ds(_al8(w * CAP + off), 128)])
        pltpu.sync_copy(dlc.at[pl.ds(0, 128)], dlc_hbm.at[pl.ds(_al8(w * CAP + off), 128)])
        cbuf[pl.ds(0, 16)] = jnp.full((16,), 1, jnp.int32) * off
        pltpu.sync_copy(cbuf, cnt_hbm.at[pl.ds(_al8(w * 16), 16)])

    return k(src, dst)


def _sc_agg(x, sel_all, dlc_all, counts):
    mesh = plsc.VectorSubcoreMesh(core_axis_name="c", subcore_axis_name="s")

    @functools.partial(
        pl.kernel,
        mesh=mesh,
        out_type=jax.ShapeDtypeStruct((NPAD, D), jnp.float32),
        scratch_types=[
            pltpu.VMEM((B,), jnp.int32),
            pltpu.VMEM((B,), jnp.int32),
            pltpu.VMEM((B + 16,), jnp.int32),
            pltpu.VMEM((B + 16,), jnp.int32),
            pltpu.VMEM((B, D), jnp.float32),
            pltpu.VMEM((B, D), jnp.float32),
            pltpu.VMEM((R + 1, D), jnp.float32),
            pltpu.VMEM((16,), jnp.int32),
            pltpu.SemaphoreType.DMA,
            pltpu.SemaphoreType.DMA,
        ],
        **_SC_PARAMS,
    )
    def k(x_hbm, sel_hbm, dlc_hbm, cnt_hbm, out_hbm, selv0, selv1, dlcv0,
          dlcv1, msgs0, msgs1, agg, cbuf, sem_i, sem_g):
        w = lax.axis_index("s") * 2 + lax.axis_index("c")
        lo = w * R
        lanes = lax.broadcasted_iota(jnp.int32, (16,), 0)
        inf16 = jnp.full((16,), jnp.inf, jnp.float32)

        def init_row(r, carry):
            for cc in range(8):
                agg[r, pl.ds(cc * 16, 16)] = inf16
            return carry

        lax.fori_loop(0, R + 1, init_row, 0)

        pltpu.sync_copy(cnt_hbm.at[pl.ds(_al8(w * 16), 16)], cbuf)
        tot = jnp.sum(cbuf[pl.ds(0, 16)]) >> 4
        nblk = jnp.maximum((tot + 127) >> 7, 1)

        def fire_idx(b, sv, dv):
            pltpu.async_copy(sel_hbm.at[pl.ds(_al8(w * CAP + b * B), B)], sv, sem_i)
            pltpu.async_copy(dlc_hbm.at[pl.ds(_al8(w * CAP + b * B), B)],
                             dv.at[pl.ds(0, B)], sem_i)

        def wait_idx(sv, dv):
            pltpu.make_async_copy(sel_hbm.at[pl.ds(0, B)], sv, sem_i).wait()
            pltpu.make_async_copy(dlc_hbm.at[pl.ds(0, B)], dv.at[pl.ds(0, B)],
                                  sem_i).wait()

        def fire_gather(sv, mb):
            pltpu.async_copy(x_hbm.at[sv], mb, sem_g)

        def wait_gather(sv, mb):
            pltpu.make_async_copy(x_hbm.at[sv], mb, sem_g).wait()

        pltpu.sync_copy(sel_hbm.at[pl.ds(_al8(w * CAP), B)], selv0)
        pltpu.sync_copy(dlc_hbm.at[pl.ds(_al8(w * CAP), B)], dlcv0.at[pl.ds(0, B)])
        fire_gather(selv0, msgs0)

        @pl.when(1 < nblk)
        def _():
            fire_idx(1, selv1, dlcv1)

        def half(b, sv, dv, mb, nsv, ndv, nmb):
            wait_gather(sv, mb)

            @pl.when(b + 1 < nblk)
            def _():
                wait_idx(nsv, ndv)
                fire_gather(nsv, nmb)

            db = dv

            def extract(i):
                g0 = (i >> 4) << 4
                dl16 = db[pl.ds(g0, 16)]
                return jnp.sum(jnp.where(lanes == i - g0, dl16, 0))

            def rmw(i, drow):
                nxt = extract(i + 1)
                for cc in range(8):
                    slc = pl.ds(cc * 16, 16)
                    agg[drow, slc] = jnp.minimum(agg[drow, slc], mb[i, slc])
                return nxt

            lax.fori_loop(0, B, rmw, extract(0))

            @pl.when(b + 2 < nblk)
            def _():
                fire_idx(b + 2, sv, dv)

        def pairb(p, carry):
            b0 = 2 * p

            @pl.when(b0 < nblk)
            def _():
                half(b0, selv0, dlcv0, msgs0, selv1, dlcv1, msgs1)

            @pl.when(b0 + 1 < nblk)
            def _():
                half(b0 + 1, selv1, dlcv1, msgs1, selv0, dlcv0, msgs0)

            return carry

        lax.fori_loop(0, (nblk + 1) >> 1, pairb, 0)
        pltpu.sync_copy(agg.at[pl.ds(0, R)], out_hbm.at[pl.ds(lo, R)])

    return k(x, sel_all, dlc_all, counts)


def _dense(agg, x, wl, b, wr, do_relu):
    BM = 1000

    def body(a_ref, x_ref, wl_ref, b_ref, wr_ref, o_ref):
        a = a_ref[...]
        a = jnp.where(jnp.isinf(a), 0.0, a)
        r = (
            jnp.dot(a, wl_ref[...], preferred_element_type=jnp.float32)
            + jnp.dot(x_ref[...], wr_ref[...], preferred_element_type=jnp.float32)
            + b_ref[...]
        )
        if do_relu:
            r = jnp.maximum(r, 0.0)
        o_ref[...] = r

    return pl.pallas_call(
        body,
        grid=(N // BM,),
        in_specs=[
            pl.BlockSpec((BM, D), lambda i: (i, 0)),
            pl.BlockSpec((BM, D), lambda i: (i, 0)),
            pl.BlockSpec((D, D), lambda i: (0, 0)),
            pl.BlockSpec((1, D), lambda i: (0, 0)),
            pl.BlockSpec((D, D), lambda i: (0, 0)),
        ],
        out_specs=pl.BlockSpec((BM, D), lambda i: (i, 0)),
        out_shape=jax.ShapeDtypeStruct((N, D), jnp.float32),
    )(agg, x, wl, b.reshape(1, D), wr)


def kernel(x, edge_index, W1_l, b1_l, W1_r, W2_l, b2_l, W2_r):
    src = edge_index[0]
    dst = edge_index[1]
    sel_all, dlc_all, counts = _scan_edges(src, dst)
    agg1 = _sc_agg(x, sel_all, dlc_all, counts)[:N]
    h = _dense(agg1, x, W1_l, b1_l, W1_r, True)
    agg2 = _sc_agg(h, sel_all, dlc_all, counts)[:N]
    out = _dense(agg2, h, W2_l, b2_l, W2_r, False)
    return out


# speedup vs baseline: 2.9987x; 1.0048x over previous
import functools

import jax
import jax.numpy as jnp
from jax import lax
from jax.experimental import pallas as pl
from jax.experimental.pallas import tpu as pltpu
from jax.experimental.pallas import tpu_sc as plsc

N = 10000
E = 320000
D = 128
NW = 32
R = 320
NPAD = NW * R
C = 3200
NCHUNK = E // C
GROUPS = C // 16
B = 128
SEL_PAD = C + 256
CAP = E + 2048

_SC_PARAMS = dict(
    compiler_params=pltpu.CompilerParams(needs_layout_passes=False),
)


def _al8(v):
    return pl.multiple_of(v, 8)


def _scan_edges(src, dst):
    mesh = plsc.VectorSubcoreMesh(core_axis_name="c", subcore_axis_name="s")

    @functools.partial(
        pl.kernel,
        mesh=mesh,
        out_type=(
            jax.ShapeDtypeStruct((NW * CAP,), jnp.int32),
            jax.ShapeDtypeStruct((NW * CAP,), jnp.int32),
            jax.ShapeDtypeStruct((NW * 16,), jnp.int32),
        ),
        scratch_types=[
            pltpu.VMEM((C,), jnp.int32),
            pltpu.VMEM((C,), jnp.int32),
            pltpu.VMEM((C,), jnp.int32),
            pltpu.VMEM((C,), jnp.int32),
            pltpu.VMEM((SEL_PAD,), jnp.int32),
            pltpu.VMEM((SEL_PAD,), jnp.int32),
            pltpu.VMEM((16,), jnp.int32),
            pltpu.SemaphoreType.DMA,
            pltpu.SemaphoreType.DMA,
        ],
        **_SC_PARAMS,
    )
    def k(src_hbm, dst_hbm, sel_hbm, dlc_hbm, cnt_hbm, srcb0, srcb1, dstb0,
          dstb1, sel, dlc, cbuf, sem_s, sem_d):
        w = lax.axis_index("s") * 2 + lax.axis_index("c")
        lo = w * R
        pad16 = jnp.full((16,), lo, jnp.int32)
        trash16 = jnp.full((16,), R, jnp.int32)

        def fire(t, sb, db):
            pltpu.async_copy(src_hbm.at[pl.ds(t * C, C)], sb, sem_s)
            pltpu.async_copy(dst_hbm.at[pl.ds(t * C, C)], db, sem_d)

        def drain(sb, db):
            pltpu.make_async_copy(src_hbm.at[pl.ds(0, C)], sb, sem_s).wait()
            pltpu.make_async_copy(dst_hbm.at[pl.ds(0, C)], db, sem_d).wait()

        fire(0, srcb0, dstb0)

        def chunk(t, off, sb, db, nsb, ndb):
            drain(sb, db)

            @pl.when(t + 1 < NCHUNK)
            def _():
                fire(t + 1, nsb, ndb)

            def scan_group(g, cv):
                d = db[pl.ds(g * 16, 16)]
                s = sb[pl.ds(g * 16, 16)]
                dl = d - lo
                m = (dl >= 0) & (dl < R)
                mi = jnp.where(m, 1, 0)
                pos = cv + plsc.cumsum(mi) - 1
                plsc.store_scatter(sel, [pos], s, mask=m)
                plsc.store_scatter(dlc, [pos], dl, mask=m)
                return cv + plsc.all_reduce_population_count(m)

            cv = lax.fori_loop(0, GROUPS, scan_group, jnp.zeros((16,), jnp.int32))
            cnt = jnp.sum(cv) >> 4
            for kk in range(8):
                sel[pl.ds(cnt + kk * 16, 16)] = pad16
                dlc[pl.ds(cnt + kk * 16, 16)] = trash16
            cnt16 = ((cnt + 15) >> 4) << 4
            nwr = (cnt16 + 127) >> 7

            def wr(kb, carry):
                pltpu.sync_copy(sel.at[pl.ds(kb * 128, 128)],
                                sel_hbm.at[pl.ds(_al8(w * CAP + off + kb * 128), 128)])
                pltpu.sync_copy(dlc.at[pl.ds(kb * 128, 128)],
                                dlc_hbm.at[pl.ds(_al8(w * CAP + off + kb * 128), 128)])
                return carry

            lax.fori_loop(0, nwr, wr, 0)
            return off + cnt16

        def pair(p, off):
            off = chunk(2 * p, off, srcb0, dstb0, srcb1, dstb1)
            off = chunk(2 * p + 1, off, srcb1, dstb1, srcb0, dstb0)
            return off

        off = lax.fori_loop(0, NCHUNK // 2, pair, 0)

        for kk in range(8):
            sel[pl.ds(kk * 16, 16)] = pad16
            dlc[pl.ds(kk * 16, 16)] = trash16
        pltpu.sync_copy(sel.at[pl.ds(0, 128)], sel_hbm.at[pl.ds(_al8(w * CAP + off), 128)])
        pltpu.sync_copy(dlc.at[pl.ds(0, 128)], dlc_hbm.at[pl.ds(_al8(w * CAP + off), 128)])
        cbuf[pl.ds(0, 16)] = jnp.full((16,), 1, jnp.int32) * off
        pltpu.sync_copy(cbuf, cnt_hbm.at[pl.ds(_al8(w * 16), 16)])

    return k(src, dst)


def _sc_agg(x, sel_all, dlc_all, counts):
    mesh = plsc.VectorSubcoreMesh(core_axis_name="c", subcore_axis_name="s")

    @functools.partial(
        pl.kernel,
        mesh=mesh,
        out_type=jax.ShapeDtypeStruct((NPAD, D), jnp.float32),
        scratch_types=[
            pltpu.VMEM((B,), jnp.int32),
            pltpu.VMEM((B,), jnp.int32),
            pltpu.VMEM((B + 16,), jnp.int32),
            pltpu.VMEM((B + 16,), jnp.int32),
            pltpu.VMEM((B, D), jnp.float32),
            pltpu.VMEM((B, D), jnp.float32),
            pltpu.VMEM((R + 1, D), jnp.float32),
            pltpu.VMEM((16,), jnp.int32),
            pltpu.SemaphoreType.DMA,
            pltpu.SemaphoreType.DMA,
        ],
        **_SC_PARAMS,
    )
    def k(x_hbm, sel_hbm, dlc_hbm, cnt_hbm, out_hbm, selv0, selv1, dlcv0,
          dlcv1, msgs0, msgs1, agg, cbuf, sem_i, sem_g):
        w = lax.axis_index("s") * 2 + lax.axis_index("c")
        lo = w * R
        lanes = lax.broadcasted_iota(jnp.int32, (16,), 0)
        inf16 = jnp.full((16,), jnp.inf, jnp.float32)

        def init_row(r, carry):
            for cc in range(8):
                agg[r, pl.ds(cc * 16, 16)] = inf16
            return carry

        lax.fori_loop(0, R + 1, init_row, 0)

        pltpu.sync_copy(cnt_hbm.at[pl.ds(_al8(w * 16), 16)], cbuf)
        tot = jnp.sum(cbuf[pl.ds(0, 16)]) >> 4
        nblk = jnp.maximum((tot + 127) >> 7, 1)

        def fire_idx(b, sv, dv):
            pltpu.async_copy(sel_hbm.at[pl.ds(_al8(w * CAP + b * B), B)], sv, sem_i)
            pltpu.async_copy(dlc_hbm.at[pl.ds(_al8(w * CAP + b * B), B)],
                             dv.at[pl.ds(0, B)], sem_i)

        def wait_idx(sv, dv):
            pltpu.make_async_copy(sel_hbm.at[pl.ds(0, B)], sv, sem_i).wait()
            pltpu.make_async_copy(dlc_hbm.at[pl.ds(0, B)], dv.at[pl.ds(0, B)],
                                  sem_i).wait()

        def fire_gather(sv, mb):
            pltpu.async_copy(x_hbm.at[sv], mb, sem_g)

        def wait_gather(sv, mb):
            pltpu.make_async_copy(x_hbm.at[sv], mb, sem_g).wait()

        pltpu.sync_copy(sel_hbm.at[pl.ds(_al8(w * CAP), B)], selv0)
        pltpu.sync_copy(dlc_hbm.at[pl.ds(_al8(w * CAP), B)], dlcv0.at[pl.ds(0, B)])
        fire_gather(selv0, msgs0)

        @pl.when(1 < nblk)
        def _():
            fire_idx(1, selv1, dlcv1)

        def half(b, sv, dv, mb, nsv, ndv, nmb):
            wait_gather(sv, mb)

            @pl.when(b + 1 < nblk)
            def _():
                wait_idx(nsv, ndv)
                fire_gather(nsv, nmb)

            db = dv

            def lane_of(v, j):
                return jnp.sum(jnp.where(lanes == j, v, 0))

            def group_rmw(g, carry):
                drow, dl16 = carry
                dl16n = db[pl.ds((g + 1) * 16, 16)]
                base = g * 16
                for j in range(16):
                    nxt = (lane_of(dl16, j + 1) if j < 15
                           else lane_of(dl16n, 0))
                    for cc in range(8):
                        slc = pl.ds(cc * 16, 16)
                        agg[drow, slc] = jnp.minimum(agg[drow, slc],
                                                     mb[base + j, slc])
                    drow = nxt
                return drow, dl16n

            dl16_0 = db[pl.ds(0, 16)]
            lax.fori_loop(0, B // 16, group_rmw, (lane_of(dl16_0, 0), dl16_0))

            @pl.when(b + 2 < nblk)
            def _():
                fire_idx(b + 2, sv, dv)

        def pairb(p, carry):
            b0 = 2 * p

            @pl.when(b0 < nblk)
            def _():
                half(b0, selv0, dlcv0, msgs0, selv1, dlcv1, msgs1)

            @pl.when(b0 + 1 < nblk)
            def _():
                half(b0 + 1, selv1, dlcv1, msgs1, selv0, dlcv0, msgs0)

            return carry

        lax.fori_loop(0, (nblk + 1) >> 1, pairb, 0)
        pltpu.sync_copy(agg.at[pl.ds(0, R)], out_hbm.at[pl.ds(lo, R)])

    return k(x, sel_all, dlc_all, counts)


def _dense(agg, x, wl, b, wr, do_relu):
    BM = 1000

    def body(a_ref, x_ref, wl_ref, b_ref, wr_ref, o_ref):
        a = a_ref[...]
        a = jnp.where(jnp.isinf(a), 0.0, a)
        r = (
            jnp.dot(a, wl_ref[...], preferred_element_type=jnp.float32)
            + jnp.dot(x_ref[...], wr_ref[...], preferred_element_type=jnp.float32)
            + b_ref[...]
        )
        if do_relu:
            r = jnp.maximum(r, 0.0)
        o_ref[...] = r

    return pl.pallas_call(
        body,
        grid=(N // BM,),
        in_specs=[
            pl.BlockSpec((BM, D), lambda i: (i, 0)),
            pl.BlockSpec((BM, D), lambda i: (i, 0)),
            pl.BlockSpec((D, D), lambda i: (0, 0)),
            pl.BlockSpec((1, D), lambda i: (0, 0)),
            pl.BlockSpec((D, D), lambda i: (0, 0)),
        ],
        out_specs=pl.BlockSpec((BM, D), lambda i: (i, 0)),
        out_shape=jax.ShapeDtypeStruct((N, D), jnp.float32),
    )(agg, x, wl, b.reshape(1, D), wr)


def kernel(x, edge_index, W1_l, b1_l, W1_r, W2_l, b2_l, W2_r):
    src = edge_index[0]
    dst = edge_index[1]
    sel_all, dlc_all, counts = _scan_edges(src, dst)
    agg1 = _sc_agg(x, sel_all, dlc_all, counts)[:N]
    h = _dense(agg1, x, W1_l, b1_l, W1_r, True)
    agg2 = _sc_agg(h, sel_all, dlc_all, counts)[:N]
    out = _dense(agg2, h, W2_l, b2_l, W2_r, False)
    return out


# speedup vs baseline: 3.0073x; 1.0029x over previous
import functools

import jax
import jax.numpy as jnp
from jax import lax
from jax.experimental import pallas as pl
from jax.experimental.pallas import tpu as pltpu
from jax.experimental.pallas import tpu_sc as plsc

N = 10000
E = 320000
D = 128
NW = 32
R = 320
NPAD = NW * R
C = 3200
NCHUNK = E // C
GROUPS = C // 16
B = 128
SEL_PAD = C + 256
CAP = E + 4096

_SC_PARAMS = dict(
    compiler_params=pltpu.CompilerParams(needs_layout_passes=False),
)


def _al8(v):
    return pl.multiple_of(v, 8)


def _scan_edges(src, dst):
    mesh = plsc.VectorSubcoreMesh(core_axis_name="c", subcore_axis_name="s")

    @functools.partial(
        pl.kernel,
        mesh=mesh,
        out_type=(
            jax.ShapeDtypeStruct((NW * CAP,), jnp.int32),
            jax.ShapeDtypeStruct((NW * CAP,), jnp.int32),
            jax.ShapeDtypeStruct((NW * 16,), jnp.int32),
        ),
        scratch_types=[
            pltpu.VMEM((C,), jnp.int32),
            pltpu.VMEM((C,), jnp.int32),
            pltpu.VMEM((C,), jnp.int32),
            pltpu.VMEM((C,), jnp.int32),
            pltpu.VMEM((SEL_PAD,), jnp.int32),
            pltpu.VMEM((SEL_PAD,), jnp.int32),
            pltpu.VMEM((16,), jnp.int32),
            pltpu.SemaphoreType.DMA,
            pltpu.SemaphoreType.DMA,
        ],
        **_SC_PARAMS,
    )
    def k(src_hbm, dst_hbm, sel_hbm, dlc_hbm, cnt_hbm, srcb0, srcb1, dstb0,
          dstb1, sel, dlc, cbuf, sem_s, sem_d):
        w = lax.axis_index("s") * 2 + lax.axis_index("c")
        lo = w * R
        pad16 = jnp.full((16,), lo, jnp.int32)
        trash16 = jnp.full((16,), R, jnp.int32)

        def fire(t, sb, db):
            pltpu.async_copy(src_hbm.at[pl.ds(t * C, C)], sb, sem_s)
            pltpu.async_copy(dst_hbm.at[pl.ds(t * C, C)], db, sem_d)

        def drain(sb, db):
            pltpu.make_async_copy(src_hbm.at[pl.ds(0, C)], sb, sem_s).wait()
            pltpu.make_async_copy(dst_hbm.at[pl.ds(0, C)], db, sem_d).wait()

        fire(0, srcb0, dstb0)

        def chunk(t, off, sb, db, nsb, ndb):
            drain(sb, db)

            @pl.when(t + 1 < NCHUNK)
            def _():
                fire(t + 1, nsb, ndb)

            def scan_group(g, cv):
                d = db[pl.ds(g * 16, 16)]
                s = sb[pl.ds(g * 16, 16)]
                dl = d - lo
                m = (dl >= 0) & (dl < R)
                mi = jnp.where(m, 1, 0)
                pos = cv + plsc.cumsum(mi) - 1
                plsc.store_scatter(sel, [pos], s, mask=m)
                plsc.store_scatter(dlc, [pos], dl, mask=m)
                return cv + plsc.all_reduce_population_count(m)

            cv = lax.fori_loop(0, GROUPS, scan_group, jnp.zeros((16,), jnp.int32))
            cnt = jnp.sum(cv) >> 4
            for kk in range(8):
                sel[pl.ds(cnt + kk * 16, 16)] = pad16
                dlc[pl.ds(cnt + kk * 16, 16)] = trash16
            cnt16 = ((cnt + 15) >> 4) << 4
            nwr = (cnt16 + 127) >> 7

            def wr(kb, carry):
                pltpu.sync_copy(sel.at[pl.ds(kb * 128, 128)],
                                sel_hbm.at[pl.ds(_al8(w * CAP + off + kb * 128), 128)])
                pltpu.sync_copy(dlc.at[pl.ds(kb * 128, 128)],
                                dlc_hbm.at[pl.ds(_al8(w * CAP + off + kb * 128), 128)])
                return carry

            lax.fori_loop(0, nwr, wr, 0)
            return off + cnt16

        def pair(p, off):
            off = chunk(2 * p, off, srcb0, dstb0, srcb1, dstb1)
            off = chunk(2 * p + 1, off, srcb1, dstb1, srcb0, dstb0)
            return off

        off = lax.fori_loop(0, NCHUNK // 2, pair, 0)

        for kk in range(8):
            sel[pl.ds(kk * 16, 16)] = pad16
            dlc[pl.ds(kk * 16, 16)] = trash16
        pltpu.sync_copy(sel.at[pl.ds(0, 128)], sel_hbm.at[pl.ds(_al8(w * CAP + off), 128)])
        pltpu.sync_copy(dlc.at[pl.ds(0, 128)], dlc_hbm.at[pl.ds(_al8(w * CAP + off), 128)])
        cbuf[pl.ds(0, 16)] = jnp.full((16,), 1, jnp.int32) * off
        pltpu.sync_copy(cbuf, cnt_hbm.at[pl.ds(_al8(w * 16), 16)])

    return k(src, dst)


def _sc_agg(x, sel_all, dlc_all, counts):
    mesh = plsc.VectorSubcoreMesh(core_axis_name="c", subcore_axis_name="s")

    @functools.partial(
        pl.kernel,
        mesh=mesh,
        out_type=jax.ShapeDtypeStruct((NPAD, D), jnp.float32),
        scratch_types=[
            pltpu.VMEM((B,), jnp.int32),
            pltpu.VMEM((B,), jnp.int32),
            pltpu.VMEM((B + 16,), jnp.int32),
            pltpu.VMEM((B + 16,), jnp.int32),
            pltpu.VMEM((B, D), jnp.float32),
            pltpu.VMEM((B, D), jnp.float32),
            pltpu.VMEM((R + 1, D), jnp.float32),
            pltpu.VMEM((16,), jnp.int32),
            pltpu.SemaphoreType.DMA,
            pltpu.SemaphoreType.DMA,
        ],
        **_SC_PARAMS,
    )
    def k(x_hbm, sel_hbm, dlc_hbm, cnt_hbm, out_hbm, selv0, selv1, dlcv0,
          dlcv1, msgs0, msgs1, agg, cbuf, sem_i, sem_g):
        w = lax.axis_index("s") * 2 + lax.axis_index("c")
        lo = w * R
        lanes = lax.broadcasted_iota(jnp.int32, (16,), 0)
        inf16 = jnp.full((16,), jnp.inf, jnp.float32)

        def init_row(r, carry):
            for cc in range(8):
                agg[r, pl.ds(cc * 16, 16)] = inf16
            return carry

        lax.fori_loop(0, R + 1, init_row, 0)

        pltpu.sync_copy(cnt_hbm.at[pl.ds(_al8(w * 16), 16)], cbuf)
        tot = jnp.sum(cbuf[pl.ds(0, 16)]) >> 4
        nblk = jnp.maximum((tot + 127) >> 7, 1)

        def fire_idx(b, sv, dv):
            pltpu.async_copy(sel_hbm.at[pl.ds(_al8(w * CAP + b * B), B)], sv, sem_i)
            pltpu.async_copy(dlc_hbm.at[pl.ds(_al8(w * CAP + b * B), B)],
                             dv.at[pl.ds(0, B)], sem_i)

        def wait_idx(sv, dv):
            pltpu.make_async_copy(sel_hbm.at[pl.ds(0, B)], sv, sem_i).wait()
            pltpu.make_async_copy(dlc_hbm.at[pl.ds(0, B)], dv.at[pl.ds(0, B)],
                                  sem_i).wait()

        def fire_gather(sv, mb):
            pltpu.async_copy(x_hbm.at[sv], mb, sem_g)

        def wait_gather(sv, mb):
            pltpu.make_async_copy(x_hbm.at[sv], mb, sem_g).wait()

        pltpu.sync_copy(sel_hbm.at[pl.ds(_al8(w * CAP), B)], selv0)
        pltpu.sync_copy(dlc_hbm.at[pl.ds(_al8(w * CAP), B)], dlcv0.at[pl.ds(0, B)])
        fire_gather(selv0, msgs0)

        @pl.when(1 < nblk)
        def _():
            fire_idx(1, selv1, dlcv1)

        def half(b, sv, dv, mb, nsv, ndv, nmb):
            wait_gather(sv, mb)

            @pl.when(b + 1 < nblk)
            def _():
                wait_idx(nsv, ndv)
                fire_gather(nsv, nmb)

            db = dv

            def lane_of(v, j):
                return jnp.sum(jnp.where(lanes == j, v, 0))

            def group_rmw(g, carry):
                drow, dl16 = carry
                dl16n = db[pl.ds((g + 1) * 16, 16)]
                base = g * 16
                for j in range(16):
                    nxt = (lane_of(dl16, j + 1) if j < 15
                           else lane_of(dl16n, 0))
                    for cc in range(8):
                        slc = pl.ds(cc * 16, 16)
                        agg[drow, slc] = jnp.minimum(agg[drow, slc],
                                                     mb[base + j, slc])
                    drow = nxt
                return drow, dl16n

            dl16_0 = db[pl.ds(0, 16)]
            lax.fori_loop(0, B // 16, group_rmw, (lane_of(dl16_0, 0), dl16_0))

            @pl.when(b + 2 < nblk)
            def _():
                fire_idx(b + 2, sv, dv)

        def pairb(p, carry):
            b0 = 2 * p

            @pl.when(b0 < nblk)
            def _():
                half(b0, selv0, dlcv0, msgs0, selv1, dlcv1, msgs1)

            @pl.when(b0 + 1 < nblk)
            def _():
                half(b0 + 1, selv1, dlcv1, msgs1, selv0, dlcv0, msgs0)

            return carry

        lax.fori_loop(0, (nblk + 1) >> 1, pairb, 0)
        pltpu.sync_copy(agg.at[pl.ds(0, R)], out_hbm.at[pl.ds(lo, R)])

    return k(x, sel_all, dlc_all, counts)


def _dense(agg, x, wl, b, wr, do_relu):
    BM = 1000

    def body(a_ref, x_ref, wl_ref, b_ref, wr_ref, o_ref):
        a = a_ref[...]
        a = jnp.where(jnp.isinf(a), 0.0, a)
        r = (
            jnp.dot(a, wl_ref[...], preferred_element_type=jnp.float32)
            + jnp.dot(x_ref[...], wr_ref[...], preferred_element_type=jnp.float32)
            + b_ref[...]
        )
        if do_relu:
            r = jnp.maximum(r, 0.0)
        o_ref[...] = r

    return pl.pallas_call(
        body,
        grid=(N // BM,),
        in_specs=[
            pl.BlockSpec((BM, D), lambda i: (i, 0)),
            pl.BlockSpec((BM, D), lambda i: (i, 0)),
            pl.BlockSpec((D, D), lambda i: (0, 0)),
            pl.BlockSpec((1, D), lambda i: (0, 0)),
            pl.BlockSpec((D, D), lambda i: (0, 0)),
        ],
        out_specs=pl.BlockSpec((BM, D), lambda i: (i, 0)),
        out_shape=jax.ShapeDtypeStruct((N, D), jnp.float32),
    )(agg, x, wl, b.reshape(1, D), wr)


def kernel(x, edge_index, W1_l, b1_l, W1_r, W2_l, b2_l, W2_r):
    src = edge_index[0]
    dst = edge_index[1]
    sel_all, dlc_all, counts = _scan_edges(src, dst)
    agg1 = _sc_agg(x, sel_all, dlc_all, counts)[:N]
    h = _dense(agg1, x, W1_l, b1_l, W1_r, True)
    agg2 = _sc_agg(h, sel_all, dlc_all, counts)[:N]
    out = _dense(agg2, h, W2_l, b2_l, W2_r, False)
    return out
